# baseline (device time: 367939 ns/iter reference)
import jax
import jax.numpy as jnp
from jax import lax
from jax.experimental import pallas as pl
from jax.experimental.pallas import tpu as pltpu

N_DEV = 16
SQ = 2048
DM = 1024
CHUNK = SQ // N_DEV


def _allreduce_body(x_ref, out_ref, staging, rs_send, rs_recv, ag_send, ag_recv):
    my = lax.axis_index("i")
    left = lax.rem(my + N_DEV - 1, N_DEV)
    right = lax.rem(my + 1, N_DEV)

    barrier = pltpu.get_barrier_semaphore()
    for nbr in (left, right):
        pl.semaphore_signal(
            barrier, inc=1, device_id=(nbr,), device_id_type=pl.DeviceIdType.MESH
        )
    pl.semaphore_wait(barrier, 2)

    out_ref[...] = x_ref[...]

    def rows(c):
        return pl.ds(c * CHUNK, CHUNK)

    for s in range(N_DEV - 1):
        send_c = lax.rem(my - s + N_DEV, N_DEV)
        rdma = pltpu.make_async_remote_copy(
            src_ref=out_ref.at[rows(send_c)],
            dst_ref=staging.at[s],
            send_sem=rs_send.at[s],
            recv_sem=rs_recv.at[s],
            device_id=(right,),
            device_id_type=pl.DeviceIdType.MESH,
        )
        rdma.start()
        rdma.wait()
        recv_c = lax.rem(my - s - 1 + N_DEV, N_DEV)
        out_ref[rows(recv_c), :] = out_ref[rows(recv_c), :] + staging[s]

    for t in range(N_DEV - 1):
        send_c = lax.rem(my + 1 - t + N_DEV, N_DEV)
        rdma = pltpu.make_async_remote_copy(
            src_ref=out_ref.at[rows(send_c)],
            dst_ref=out_ref.at[rows(send_c)],
            send_sem=ag_send.at[t],
            recv_sem=ag_recv.at[t],
            device_id=(right,),
            device_id_type=pl.DeviceIdType.MESH,
        )
        rdma.start()
        rdma.wait()


def _ring_allreduce(partial):
    return pl.pallas_call(
        _allreduce_body,
        out_shape=jax.ShapeDtypeStruct((SQ, DM), jnp.float32),
        in_specs=[pl.BlockSpec(memory_space=pltpu.VMEM)],
        out_specs=pl.BlockSpec(memory_space=pltpu.VMEM),
        scratch_shapes=[
            pltpu.VMEM((N_DEV - 1, CHUNK, DM), jnp.float32),
            pltpu.SemaphoreType.DMA((N_DEV - 1,)),
            pltpu.SemaphoreType.DMA((N_DEV - 1,)),
            pltpu.SemaphoreType.DMA((N_DEV - 1,)),
            pltpu.SemaphoreType.DMA((N_DEV - 1,)),
        ],
        compiler_params=pltpu.CompilerParams(collective_id=0),
    )(partial)


def kernel(x, Wq, K_ext, V_ext, Wo):
    my = lax.axis_index("i")
    sq = x.shape[1]
    hq_per = K_ext.shape[2]
    dh = K_ext.shape[3]
    dcols = hq_per * dh

    Wq_l = lax.dynamic_slice(Wq, (0, my * dcols), (Wq.shape[0], dcols))
    Wo_l = lax.dynamic_slice(Wo, (my * dcols, 0), (dcols, Wo.shape[1]))

    xb = x[0].astype(jnp.bfloat16)
    Q = jnp.dot(xb, Wq_l.astype(jnp.bfloat16), preferred_element_type=jnp.float32)
    Q = Q.reshape(sq, hq_per, dh)

    K = K_ext[0].astype(jnp.bfloat16)
    V = V_ext[0].astype(jnp.bfloat16)

    scores = jnp.einsum(
        "ihd,jhd->hij", Q.astype(jnp.bfloat16), K,
        preferred_element_type=jnp.float32,
    ) * 0.08838834764831843
    qi = lax.broadcasted_iota(jnp.int32, (sq, sq), 0)
    ki = lax.broadcasted_iota(jnp.int32, (sq, sq), 1)
    mask = (jnp.abs(qi - ki) <= 128) | (ki < 32) | (qi < 32)
    scores = jnp.where(mask[None], scores, -1e9)
    m = jnp.max(scores, axis=-1, keepdims=True)
    w = jnp.exp(scores - m)
    w = w / jnp.sum(w, axis=-1, keepdims=True)

    ctx = jnp.einsum(
        "hij,jhd->ihd", w.astype(jnp.bfloat16), V,
        preferred_element_type=jnp.float32,
    ).reshape(sq, dcols)

    partial = jnp.dot(
        ctx.astype(jnp.bfloat16), Wo_l.astype(jnp.bfloat16),
        preferred_element_type=jnp.float32,
    )

    out = _ring_allreduce(partial)
    return out[None]


# device time: 282636 ns/iter; 1.3018x vs baseline; 1.3018x over previous
import jax
import jax.numpy as jnp
from jax import lax
from jax.experimental import pallas as pl
from jax.experimental.pallas import tpu as pltpu

N_DEV = 16
SQ = 2048
DM = 1024
CHUNK = SQ // N_DEV


def _allreduce_body(x_ref, out_ref, staging, rs_send, rs_recv, ag_send, ag_recv):
    my = lax.axis_index("i")
    left = lax.rem(my + N_DEV - 1, N_DEV)
    right = lax.rem(my + 1, N_DEV)

    barrier = pltpu.get_barrier_semaphore()
    for nbr in (left, right):
        pl.semaphore_signal(
            barrier, inc=1, device_id=(nbr,), device_id_type=pl.DeviceIdType.MESH
        )
    pl.semaphore_wait(barrier, 2)

    out_ref[...] = x_ref[...]

    def rows(c):
        return pl.ds(c * CHUNK, CHUNK)

    for s in range(N_DEV - 1):
        send_c = lax.rem(my - s + N_DEV, N_DEV)
        rdma = pltpu.make_async_remote_copy(
            src_ref=out_ref.at[rows(send_c)],
            dst_ref=staging.at[s],
            send_sem=rs_send.at[s],
            recv_sem=rs_recv.at[s],
            device_id=(right,),
            device_id_type=pl.DeviceIdType.MESH,
        )
        rdma.start()
        rdma.wait()
        recv_c = lax.rem(my - s - 1 + N_DEV, N_DEV)
        out_ref[rows(recv_c), :] = out_ref[rows(recv_c), :] + staging[s]

    for t in range(N_DEV - 1):
        send_c = lax.rem(my + 1 - t + N_DEV, N_DEV)
        rdma = pltpu.make_async_remote_copy(
            src_ref=out_ref.at[rows(send_c)],
            dst_ref=out_ref.at[rows(send_c)],
            send_sem=ag_send.at[t],
            recv_sem=ag_recv.at[t],
            device_id=(right,),
            device_id_type=pl.DeviceIdType.MESH,
        )
        rdma.start()
        rdma.wait()


def _ring_allreduce(partial):
    return pl.pallas_call(
        _allreduce_body,
        out_shape=jax.ShapeDtypeStruct((SQ, DM), jnp.bfloat16),
        in_specs=[pl.BlockSpec(memory_space=pltpu.VMEM)],
        out_specs=pl.BlockSpec(memory_space=pltpu.VMEM),
        scratch_shapes=[
            pltpu.VMEM((N_DEV - 1, CHUNK, DM), jnp.bfloat16),
            pltpu.SemaphoreType.DMA((N_DEV - 1,)),
            pltpu.SemaphoreType.DMA((N_DEV - 1,)),
            pltpu.SemaphoreType.DMA((N_DEV - 1,)),
            pltpu.SemaphoreType.DMA((N_DEV - 1,)),
        ],
        compiler_params=pltpu.CompilerParams(collective_id=0),
    )(partial)


def kernel(x, Wq, K_ext, V_ext, Wo):
    my = lax.axis_index("i")
    sq = x.shape[1]
    hq_per = K_ext.shape[2]
    dh = K_ext.shape[3]
    dcols = hq_per * dh

    Wq_l = lax.dynamic_slice(Wq, (0, my * dcols), (Wq.shape[0], dcols))
    Wo_l = lax.dynamic_slice(Wo, (my * dcols, 0), (dcols, Wo.shape[1]))

    xb = x[0].astype(jnp.bfloat16)
    Q = jnp.dot(xb, Wq_l.astype(jnp.bfloat16), preferred_element_type=jnp.float32)
    Q = Q.reshape(sq, hq_per, dh)

    K = K_ext[0].astype(jnp.bfloat16)
    V = V_ext[0].astype(jnp.bfloat16)

    scores = jnp.einsum(
        "ihd,jhd->hij", Q.astype(jnp.bfloat16), K,
        preferred_element_type=jnp.float32,
    ) * 0.08838834764831843
    qi = lax.broadcasted_iota(jnp.int32, (sq, sq), 0)
    ki = lax.broadcasted_iota(jnp.int32, (sq, sq), 1)
    mask = (jnp.abs(qi - ki) <= 128) | (ki < 32) | (qi < 32)
    scores = jnp.where(mask[None], scores, -1e9)
    m = jnp.max(scores, axis=-1, keepdims=True)
    w = jnp.exp(scores - m)
    w = w / jnp.sum(w, axis=-1, keepdims=True)

    ctx = jnp.einsum(
        "hij,jhd->ihd", w.astype(jnp.bfloat16), V,
        preferred_element_type=jnp.float32,
    ).reshape(sq, dcols)

    partial = jnp.dot(
        ctx.astype(jnp.bfloat16), Wo_l.astype(jnp.bfloat16),
        preferred_element_type=jnp.float32,
    )

    out = _ring_allreduce(partial.astype(jnp.bfloat16))
    return out.astype(jnp.float32)[None]


# device time: 246333 ns/iter; 1.4937x vs baseline; 1.1474x over previous
import jax
import jax.numpy as jnp
from jax import lax
from jax.experimental import pallas as pl
from jax.experimental.pallas import tpu as pltpu

N_DEV = 16
SQ = 2048
DM = 1024
CHUNK = SQ // N_DEV


def _allreduce_body(x_ref, out_ref, staging, rs_send, rs_recv, ag_send, ag_recv):
    my = lax.axis_index("i")
    left = lax.rem(my + N_DEV - 1, N_DEV)
    right = lax.rem(my + 1, N_DEV)

    barrier = pltpu.get_barrier_semaphore()
    for nbr in (left, right):
        pl.semaphore_signal(
            barrier, inc=1, device_id=(nbr,), device_id_type=pl.DeviceIdType.MESH
        )
    pl.semaphore_wait(barrier, 2)

    out_ref[...] = x_ref[...]

    def rows(c):
        return pl.ds(c * CHUNK, CHUNK)

    for s in range(N_DEV - 1):
        send_c = lax.rem(my - s + N_DEV, N_DEV)
        rdma = pltpu.make_async_remote_copy(
            src_ref=out_ref.at[rows(send_c)],
            dst_ref=staging.at[s],
            send_sem=rs_send.at[s],
            recv_sem=rs_recv.at[s],
            device_id=(right,),
            device_id_type=pl.DeviceIdType.MESH,
        )
        rdma.start()
        rdma.wait()
        recv_c = lax.rem(my - s - 1 + N_DEV, N_DEV)
        out_ref[rows(recv_c), :] = out_ref[rows(recv_c), :] + staging[s]

    for t in range(N_DEV - 1):
        send_c = lax.rem(my + 1 - t + N_DEV, N_DEV)
        rdma = pltpu.make_async_remote_copy(
            src_ref=out_ref.at[rows(send_c)],
            dst_ref=out_ref.at[rows(send_c)],
            send_sem=ag_send.at[t],
            recv_sem=ag_recv.at[t],
            device_id=(right,),
            device_id_type=pl.DeviceIdType.MESH,
        )
        rdma.start()
        rdma.wait()


def _ring_allreduce(partial):
    return pl.pallas_call(
        _allreduce_body,
        out_shape=jax.ShapeDtypeStruct((SQ, DM), jnp.bfloat16),
        in_specs=[pl.BlockSpec(memory_space=pltpu.VMEM)],
        out_specs=pl.BlockSpec(memory_space=pltpu.VMEM),
        scratch_shapes=[
            pltpu.VMEM((N_DEV - 1, CHUNK, DM), jnp.bfloat16),
            pltpu.SemaphoreType.DMA((N_DEV - 1,)),
            pltpu.SemaphoreType.DMA((N_DEV - 1,)),
            pltpu.SemaphoreType.DMA((N_DEV - 1,)),
            pltpu.SemaphoreType.DMA((N_DEV - 1,)),
        ],
        compiler_params=pltpu.CompilerParams(collective_id=0),
    )(partial)


SCALE = 0.08838834764831843
QBLK = 128
WIN = 384
NEG = -1e9


def _attn_body(q_ref, k_ref, v_ref, o_ref):
    qb = pl.program_id(1)
    q = q_ref[0]

    row = lax.broadcasted_iota(jnp.int32, (QBLK, 1), 0) + qb * QBLK

    @pl.when(qb == 0)
    def _dense():
        k = k_ref[0]
        s = lax.dot_general(
            q, k, (((1,), (1,)), ((), ())), preferred_element_type=jnp.float32
        ) * SCALE
        ki = lax.broadcasted_iota(jnp.int32, (QBLK, SQ), 1)
        mask = (jnp.abs(row - ki) <= 128) | (ki < 32) | (row < 32)
        s = jnp.where(mask, s, NEG)
        m = jnp.max(s, axis=-1, keepdims=True)
        w = jnp.exp(s - m)
        denom = jnp.sum(w, axis=-1, keepdims=True)
        ctx = lax.dot_general(
            w.astype(jnp.bfloat16), v_ref[0], (((1,), (0,)), ((), ())),
            preferred_element_type=jnp.float32,
        )
        o_ref[0] = (ctx / denom).astype(jnp.bfloat16)

    @pl.when(qb > 0)
    def _band():
        ws = pl.multiple_of(jnp.clip((qb - 1) * QBLK, 0, SQ - WIN), QBLK)
        kw = k_ref[0, pl.ds(ws, WIN), :]
        vw = v_ref[0, pl.ds(ws, WIN), :]
        k0 = k_ref[0, :QBLK, :]
        v0 = v_ref[0, :QBLK, :]

        sb = lax.dot_general(
            q, kw, (((1,), (1,)), ((), ())), preferred_element_type=jnp.float32
        ) * SCALE
        kib = lax.broadcasted_iota(jnp.int32, (QBLK, WIN), 1) + ws
        mb = (jnp.abs(row - kib) <= 128) | (kib < 32)
        sb = jnp.where(mb, sb, NEG)

        sg = lax.dot_general(
            q, k0, (((1,), (1,)), ((), ())), preferred_element_type=jnp.float32
        ) * SCALE
        kig = lax.broadcasted_iota(jnp.int32, (QBLK, QBLK), 1)
        mg = (kig < 32) & (qb >= 2)
        sg = jnp.where(mg, sg, NEG)

        m = jnp.maximum(
            jnp.max(sb, axis=-1, keepdims=True),
            jnp.max(sg, axis=-1, keepdims=True),
        )
        wb = jnp.exp(sb - m)
        wg = jnp.exp(sg - m)
        denom = jnp.sum(wb, axis=-1, keepdims=True) + jnp.sum(
            wg, axis=-1, keepdims=True
        )
        ctx = lax.dot_general(
            wb.astype(jnp.bfloat16), vw, (((1,), (0,)), ((), ())),
            preferred_element_type=jnp.float32,
        ) + lax.dot_general(
            wg.astype(jnp.bfloat16), v0, (((1,), (0,)), ((), ())),
            preferred_element_type=jnp.float32,
        )
        o_ref[0] = (ctx / denom).astype(jnp.bfloat16)


def _sparse_attn(q_hm, k_hm, v_hm, hq_per):
    return pl.pallas_call(
        _attn_body,
        grid=(hq_per, SQ // QBLK),
        out_shape=jax.ShapeDtypeStruct((hq_per, SQ, 128), jnp.bfloat16),
        in_specs=[
            pl.BlockSpec((1, QBLK, 128), lambda h, qb: (h, qb, 0)),
            pl.BlockSpec((1, SQ, 128), lambda h, qb: (h, 0, 0)),
            pl.BlockSpec((1, SQ, 128), lambda h, qb: (h, 0, 0)),
        ],
        out_specs=pl.BlockSpec((1, QBLK, 128), lambda h, qb: (h, qb, 0)),
    )(q_hm, k_hm, v_hm)


def kernel(x, Wq, K_ext, V_ext, Wo):
    my = lax.axis_index("i")
    sq = x.shape[1]
    hq_per = K_ext.shape[2]
    dh = K_ext.shape[3]
    dcols = hq_per * dh

    Wq_l = lax.dynamic_slice(Wq, (0, my * dcols), (Wq.shape[0], dcols))
    Wo_l = lax.dynamic_slice(Wo, (my * dcols, 0), (dcols, Wo.shape[1]))

    xb = x[0].astype(jnp.bfloat16)
    Q = jnp.dot(xb, Wq_l.astype(jnp.bfloat16), preferred_element_type=jnp.float32)
    q_hm = Q.reshape(sq, hq_per, dh).transpose(1, 0, 2).astype(jnp.bfloat16)
    k_hm = K_ext[0].transpose(1, 0, 2).astype(jnp.bfloat16)
    v_hm = V_ext[0].transpose(1, 0, 2).astype(jnp.bfloat16)

    ctx = _sparse_attn(q_hm, k_hm, v_hm, hq_per)
    ctx = ctx.transpose(1, 0, 2).reshape(sq, dcols)

    partial = jnp.dot(
        ctx, Wo_l.astype(jnp.bfloat16), preferred_element_type=jnp.float32
    )

    out = _ring_allreduce(partial.astype(jnp.bfloat16))
    return out.astype(jnp.float32)[None]


# device time: 243145 ns/iter; 1.5132x vs baseline; 1.0131x over previous
import jax
import jax.numpy as jnp
from jax import lax
from jax.experimental import pallas as pl
from jax.experimental.pallas import tpu as pltpu

N_DEV = 16
SQ = 2048
DM = 1024
CHUNK = SQ // N_DEV


HALF = DM // 2


def _allreduce_body(
    x_ref, out_ref, stag_r, stag_l,
    rs_send_r, rs_recv_r, ag_send_r, ag_recv_r,
    rs_send_l, rs_recv_l, ag_send_l, ag_recv_l,
):
    my = lax.axis_index("i")
    left = lax.rem(my + N_DEV - 1, N_DEV)
    right = lax.rem(my + 1, N_DEV)

    barrier = pltpu.get_barrier_semaphore()
    for nbr in (left, right):
        pl.semaphore_signal(
            barrier, inc=1, device_id=(nbr,), device_id_type=pl.DeviceIdType.MESH
        )
    pl.semaphore_wait(barrier, 2)

    out_ref[...] = x_ref[...]

    def rows(c):
        return pl.ds(c * CHUNK, CHUNK)


    for s in range(N_DEV - 1):
        send_r = lax.rem(my - s + N_DEV, N_DEV)
        rdma_r = pltpu.make_async_remote_copy(
            src_ref=out_ref.at[rows(send_r), pl.ds(0, HALF)],
            dst_ref=stag_r.at[s],
            send_sem=rs_send_r.at[s],
            recv_sem=rs_recv_r.at[s],
            device_id=(right,),
            device_id_type=pl.DeviceIdType.MESH,
        )
        send_l = lax.rem(my + s, N_DEV)
        rdma_l = pltpu.make_async_remote_copy(
            src_ref=out_ref.at[rows(send_l), pl.ds(HALF, HALF)],
            dst_ref=stag_l.at[s],
            send_sem=rs_send_l.at[s],
            recv_sem=rs_recv_l.at[s],
            device_id=(left,),
            device_id_type=pl.DeviceIdType.MESH,
        )
        rdma_r.start()
        rdma_l.start()
        rdma_r.wait()
        recv_r = lax.rem(my - s - 1 + N_DEV, N_DEV)
        out_ref[rows(recv_r), :HALF] = out_ref[rows(recv_r), :HALF] + stag_r[s]
        rdma_l.wait()
        recv_l = lax.rem(my + s + 1, N_DEV)
        out_ref[rows(recv_l), HALF:] = out_ref[rows(recv_l), HALF:] + stag_l[s]

    for t in range(N_DEV - 1):
        send_r = lax.rem(my + 1 - t + N_DEV, N_DEV)
        rdma_r = pltpu.make_async_remote_copy(
            src_ref=out_ref.at[rows(send_r), pl.ds(0, HALF)],
            dst_ref=out_ref.at[rows(send_r), pl.ds(0, HALF)],
            send_sem=ag_send_r.at[t],
            recv_sem=ag_recv_r.at[t],
            device_id=(right,),
            device_id_type=pl.DeviceIdType.MESH,
        )
        send_l = lax.rem(my - 1 + t + N_DEV, N_DEV)
        rdma_l = pltpu.make_async_remote_copy(
            src_ref=out_ref.at[rows(send_l), pl.ds(HALF, HALF)],
            dst_ref=out_ref.at[rows(send_l), pl.ds(HALF, HALF)],
            send_sem=ag_send_l.at[t],
            recv_sem=ag_recv_l.at[t],
            device_id=(left,),
            device_id_type=pl.DeviceIdType.MESH,
        )
        rdma_r.start()
        rdma_l.start()
        rdma_r.wait()
        rdma_l.wait()


def _ring_allreduce(partial):
    return pl.pallas_call(
        _allreduce_body,
        out_shape=jax.ShapeDtypeStruct((SQ, DM), jnp.bfloat16),
        in_specs=[pl.BlockSpec(memory_space=pltpu.VMEM)],
        out_specs=pl.BlockSpec(memory_space=pltpu.VMEM),
        scratch_shapes=[
            pltpu.VMEM((N_DEV - 1, CHUNK, HALF), jnp.bfloat16),
            pltpu.VMEM((N_DEV - 1, CHUNK, HALF), jnp.bfloat16),
        ]
        + [pltpu.SemaphoreType.DMA((N_DEV - 1,)) for _ in range(8)],
        compiler_params=pltpu.CompilerParams(collective_id=0),
    )(partial)


SCALE = 0.08838834764831843
QBLK = 128
WIN = 384
NEG = -1e9


def _attn_body(q_ref, k_ref, v_ref, o_ref):
    qb = pl.program_id(1)
    q = q_ref[0]

    row = lax.broadcasted_iota(jnp.int32, (QBLK, 1), 0) + qb * QBLK

    @pl.when(qb == 0)
    def _dense():
        k = k_ref[0]
        s = lax.dot_general(
            q, k, (((1,), (1,)), ((), ())), preferred_element_type=jnp.float32
        ) * SCALE
        ki = lax.broadcasted_iota(jnp.int32, (QBLK, SQ), 1)
        mask = (jnp.abs(row - ki) <= 128) | (ki < 32) | (row < 32)
        s = jnp.where(mask, s, NEG)
        m = jnp.max(s, axis=-1, keepdims=True)
        w = jnp.exp(s - m)
        denom = jnp.sum(w, axis=-1, keepdims=True)
        ctx = lax.dot_general(
            w.astype(jnp.bfloat16), v_ref[0], (((1,), (0,)), ((), ())),
            preferred_element_type=jnp.float32,
        )
        o_ref[0] = (ctx / denom).astype(jnp.bfloat16)

    @pl.when(qb > 0)
    def _band():
        ws = pl.multiple_of(jnp.clip((qb - 1) * QBLK, 0, SQ - WIN), QBLK)
        kw = k_ref[0, pl.ds(ws, WIN), :]
        vw = v_ref[0, pl.ds(ws, WIN), :]
        k0 = k_ref[0, :QBLK, :]
        v0 = v_ref[0, :QBLK, :]

        sb = lax.dot_general(
            q, kw, (((1,), (1,)), ((), ())), preferred_element_type=jnp.float32
        ) * SCALE
        kib = lax.broadcasted_iota(jnp.int32, (QBLK, WIN), 1) + ws
        mb = (jnp.abs(row - kib) <= 128) | (kib < 32)
        sb = jnp.where(mb, sb, NEG)

        sg = lax.dot_general(
            q, k0, (((1,), (1,)), ((), ())), preferred_element_type=jnp.float32
        ) * SCALE
        kig = lax.broadcasted_iota(jnp.int32, (QBLK, QBLK), 1)
        mg = (kig < 32) & (qb >= 2)
        sg = jnp.where(mg, sg, NEG)

        m = jnp.maximum(
            jnp.max(sb, axis=-1, keepdims=True),
            jnp.max(sg, axis=-1, keepdims=True),
        )
        wb = jnp.exp(sb - m)
        wg = jnp.exp(sg - m)
        denom = jnp.sum(wb, axis=-1, keepdims=True) + jnp.sum(
            wg, axis=-1, keepdims=True
        )
        ctx = lax.dot_general(
            wb.astype(jnp.bfloat16), vw, (((1,), (0,)), ((), ())),
            preferred_element_type=jnp.float32,
        ) + lax.dot_general(
            wg.astype(jnp.bfloat16), v0, (((1,), (0,)), ((), ())),
            preferred_element_type=jnp.float32,
        )
        o_ref[0] = (ctx / denom).astype(jnp.bfloat16)


def _sparse_attn(q_hm, k_hm, v_hm, hq_per):
    return pl.pallas_call(
        _attn_body,
        grid=(hq_per, SQ // QBLK),
        out_shape=jax.ShapeDtypeStruct((hq_per, SQ, 128), jnp.bfloat16),
        in_specs=[
            pl.BlockSpec((1, QBLK, 128), lambda h, qb: (h, qb, 0)),
            pl.BlockSpec((1, SQ, 128), lambda h, qb: (h, 0, 0)),
            pl.BlockSpec((1, SQ, 128), lambda h, qb: (h, 0, 0)),
        ],
        out_specs=pl.BlockSpec((1, QBLK, 128), lambda h, qb: (h, qb, 0)),
    )(q_hm, k_hm, v_hm)


def kernel(x, Wq, K_ext, V_ext, Wo):
    my = lax.axis_index("i")
    sq = x.shape[1]
    hq_per = K_ext.shape[2]
    dh = K_ext.shape[3]
    dcols = hq_per * dh

    Wq_l = lax.dynamic_slice(Wq, (0, my * dcols), (Wq.shape[0], dcols))
    Wo_l = lax.dynamic_slice(Wo, (my * dcols, 0), (dcols, Wo.shape[1]))

    xb = x[0].astype(jnp.bfloat16)
    Q = jnp.dot(xb, Wq_l.astype(jnp.bfloat16), preferred_element_type=jnp.float32)
    q_hm = Q.reshape(sq, hq_per, dh).transpose(1, 0, 2).astype(jnp.bfloat16)
    k_hm = K_ext[0].transpose(1, 0, 2).astype(jnp.bfloat16)
    v_hm = V_ext[0].transpose(1, 0, 2).astype(jnp.bfloat16)

    ctx = _sparse_attn(q_hm, k_hm, v_hm, hq_per)
    ctx = ctx.transpose(1, 0, 2).reshape(sq, dcols)

    partial = jnp.dot(
        ctx, Wo_l.astype(jnp.bfloat16), preferred_element_type=jnp.float32
    )

    out = _ring_allreduce(partial.astype(jnp.bfloat16))
    return out.astype(jnp.float32)[None]


# device time: 180799 ns/iter; 2.0351x vs baseline; 1.3448x over previous
import jax
import jax.numpy as jnp
from jax import lax
from jax.experimental import pallas as pl
from jax.experimental.pallas import tpu as pltpu

N_DEV = 16
SQ = 2048
DM = 1024
CHUNK = SQ // N_DEV


HALF = DM // 2
QROWS = SQ // 4
SUB = SQ // 16


def _allreduce_body(
    x_ref, out_ref, stagA_r, stagA_l, stagB_r, stagB_l,
    sA_r, rA_r, sA_l, rA_l,
    sB_r, rB_r, sB_l, rB_l,
    sC_r, rC_r, sC_l, rC_l,
):
    my = lax.axis_index("i")
    zi = my // 4
    pi = lax.rem(my, 4)
    plane_r = zi * 4 + lax.rem(pi + 1, 4)
    plane_l = zi * 4 + lax.rem(pi + 3, 4)
    z_r = lax.rem(zi + 1, 4) * 4 + pi
    z_l = lax.rem(zi + 3, 4) * 4 + pi

    barrier = pltpu.get_barrier_semaphore()
    for nbr in (plane_l, plane_r, z_l, z_r):
        pl.semaphore_signal(
            barrier, inc=1, device_id=(nbr,), device_id_type=pl.DeviceIdType.MESH
        )
    pl.semaphore_wait(barrier, 4)

    out_ref[...] = x_ref[...]

    def qrows(q):
        return pl.ds(pl.multiple_of(q * QROWS, QROWS), QROWS)

    def srows(a, j):
        return pl.ds(pl.multiple_of(a * QROWS + j * SUB, SUB), SUB)

    CW = pl.ds(0, HALF)
    CCW = pl.ds(HALF, HALF)

    def rdma(src, dst, ssem, rsem, dev):
        return pltpu.make_async_remote_copy(
            src_ref=src, dst_ref=dst, send_sem=ssem, recv_sem=rsem,
            device_id=(dev,), device_id_type=pl.DeviceIdType.MESH,
        )

    for s in range(3):
        cw = rdma(out_ref.at[qrows(lax.rem(pi - s + 4, 4)), CW],
                  stagA_r.at[s], sA_r.at[s], rA_r.at[s], plane_r)
        cc = rdma(out_ref.at[qrows(lax.rem(pi + s, 4)), CCW],
                  stagA_l.at[s], sA_l.at[s], rA_l.at[s], plane_l)
        cw.start()
        cc.start()
        cw.wait()
        qa = lax.rem(pi - s - 1 + 4, 4)
        out_ref[qrows(qa), :HALF] = out_ref[qrows(qa), :HALF] + stagA_r[s]
        cc.wait()
        qb = lax.rem(pi + s + 1, 4)
        out_ref[qrows(qb), HALF:] = out_ref[qrows(qb), HALF:] + stagA_l[s]

    a_r = lax.rem(pi + 1, 4)
    a_l = lax.rem(pi + 3, 4)

    for s in range(3):
        cw = rdma(out_ref.at[srows(a_r, lax.rem(zi - s + 4, 4)), CW],
                  stagB_r.at[s], sB_r.at[s], rB_r.at[s], z_r)
        cc = rdma(out_ref.at[srows(a_l, lax.rem(zi + s, 4)), CCW],
                  stagB_l.at[s], sB_l.at[s], rB_l.at[s], z_l)
        cw.start()
        cc.start()
        cw.wait()
        ja = lax.rem(zi - s - 1 + 4, 4)
        out_ref[srows(a_r, ja), :HALF] = (
            out_ref[srows(a_r, ja), :HALF] + stagB_r[s]
        )
        cc.wait()
        jb = lax.rem(zi + s + 1, 4)
        out_ref[srows(a_l, jb), HALF:] = (
            out_ref[srows(a_l, jb), HALF:] + stagB_l[s]
        )
    for t in range(3):
        src_r = out_ref.at[srows(a_r, lax.rem(zi + 1 - t + 4, 4)), CW]
        cw = rdma(src_r, src_r, sB_r.at[3 + t], rB_r.at[3 + t], z_r)
        src_l = out_ref.at[srows(a_l, lax.rem(zi + 3 + t, 4)), CCW]
        cc = rdma(src_l, src_l, sB_l.at[3 + t], rB_l.at[3 + t], z_l)
        cw.start()
        cc.start()
        cw.wait()
        cc.wait()

    for t in range(3):
        src_r = out_ref.at[qrows(lax.rem(pi + 1 - t + 4, 4)), CW]
        cw = rdma(src_r, src_r, sC_r.at[t], rC_r.at[t], plane_r)
        src_l = out_ref.at[qrows(lax.rem(pi + 3 + t, 4)), CCW]
        cc = rdma(src_l, src_l, sC_l.at[t], rC_l.at[t], plane_l)
        cw.start()
        cc.start()
        cw.wait()
        cc.wait()


def _ring_allreduce(partial):
    return pl.pallas_call(
        _allreduce_body,
        out_shape=jax.ShapeDtypeStruct((SQ, DM), jnp.bfloat16),
        in_specs=[pl.BlockSpec(memory_space=pltpu.VMEM)],
        out_specs=pl.BlockSpec(memory_space=pltpu.VMEM),
        scratch_shapes=[
            pltpu.VMEM((3, QROWS, HALF), jnp.bfloat16),
            pltpu.VMEM((3, QROWS, HALF), jnp.bfloat16),
            pltpu.VMEM((3, SUB, HALF), jnp.bfloat16),
            pltpu.VMEM((3, SUB, HALF), jnp.bfloat16),
            pltpu.SemaphoreType.DMA((3,)),
            pltpu.SemaphoreType.DMA((3,)),
            pltpu.SemaphoreType.DMA((3,)),
            pltpu.SemaphoreType.DMA((3,)),
            pltpu.SemaphoreType.DMA((6,)),
            pltpu.SemaphoreType.DMA((6,)),
            pltpu.SemaphoreType.DMA((6,)),
            pltpu.SemaphoreType.DMA((6,)),
            pltpu.SemaphoreType.DMA((3,)),
            pltpu.SemaphoreType.DMA((3,)),
            pltpu.SemaphoreType.DMA((3,)),
            pltpu.SemaphoreType.DMA((3,)),
        ],
        compiler_params=pltpu.CompilerParams(collective_id=0),
    )(partial)


SCALE = 0.08838834764831843
QBLK = 128
WIN = 384
NEG = -1e9


def _attn_body(q_ref, k_ref, v_ref, o_ref):
    qb = pl.program_id(1)
    q = q_ref[0]

    row = lax.broadcasted_iota(jnp.int32, (QBLK, 1), 0) + qb * QBLK

    @pl.when(qb == 0)
    def _dense():
        k = k_ref[0]
        s = lax.dot_general(
            q, k, (((1,), (1,)), ((), ())), preferred_element_type=jnp.float32
        ) * SCALE
        ki = lax.broadcasted_iota(jnp.int32, (QBLK, SQ), 1)
        mask = (jnp.abs(row - ki) <= 128) | (ki < 32) | (row < 32)
        s = jnp.where(mask, s, NEG)
        m = jnp.max(s, axis=-1, keepdims=True)
        w = jnp.exp(s - m)
        denom = jnp.sum(w, axis=-1, keepdims=True)
        ctx = lax.dot_general(
            w.astype(jnp.bfloat16), v_ref[0], (((1,), (0,)), ((), ())),
            preferred_element_type=jnp.float32,
        )
        o_ref[0] = (ctx / denom).astype(jnp.bfloat16)

    @pl.when(qb > 0)
    def _band():
        ws = pl.multiple_of(jnp.clip((qb - 1) * QBLK, 0, SQ - WIN), QBLK)
        kw = k_ref[0, pl.ds(ws, WIN), :]
        vw = v_ref[0, pl.ds(ws, WIN), :]
        k0 = k_ref[0, :QBLK, :]
        v0 = v_ref[0, :QBLK, :]

        sb = lax.dot_general(
            q, kw, (((1,), (1,)), ((), ())), preferred_element_type=jnp.float32
        ) * SCALE
        kib = lax.broadcasted_iota(jnp.int32, (QBLK, WIN), 1) + ws
        mb = (jnp.abs(row - kib) <= 128) | (kib < 32)
        sb = jnp.where(mb, sb, NEG)

        sg = lax.dot_general(
            q, k0, (((1,), (1,)), ((), ())), preferred_element_type=jnp.float32
        ) * SCALE
        kig = lax.broadcasted_iota(jnp.int32, (QBLK, QBLK), 1)
        mg = (kig < 32) & (qb >= 2)
        sg = jnp.where(mg, sg, NEG)

        m = jnp.maximum(
            jnp.max(sb, axis=-1, keepdims=True),
            jnp.max(sg, axis=-1, keepdims=True),
        )
        wb = jnp.exp(sb - m)
        wg = jnp.exp(sg - m)
        denom = jnp.sum(wb, axis=-1, keepdims=True) + jnp.sum(
            wg, axis=-1, keepdims=True
        )
        ctx = lax.dot_general(
            wb.astype(jnp.bfloat16), vw, (((1,), (0,)), ((), ())),
            preferred_element_type=jnp.float32,
        ) + lax.dot_general(
            wg.astype(jnp.bfloat16), v0, (((1,), (0,)), ((), ())),
            preferred_element_type=jnp.float32,
        )
        o_ref[0] = (ctx / denom).astype(jnp.bfloat16)


def _sparse_attn(q_hm, k_hm, v_hm, hq_per):
    return pl.pallas_call(
        _attn_body,
        grid=(hq_per, SQ // QBLK),
        out_shape=jax.ShapeDtypeStruct((hq_per, SQ, 128), jnp.bfloat16),
        in_specs=[
            pl.BlockSpec((1, QBLK, 128), lambda h, qb: (h, qb, 0)),
            pl.BlockSpec((1, SQ, 128), lambda h, qb: (h, 0, 0)),
            pl.BlockSpec((1, SQ, 128), lambda h, qb: (h, 0, 0)),
        ],
        out_specs=pl.BlockSpec((1, QBLK, 128), lambda h, qb: (h, qb, 0)),
    )(q_hm, k_hm, v_hm)


def kernel(x, Wq, K_ext, V_ext, Wo):
    my = lax.axis_index("i")
    sq = x.shape[1]
    hq_per = K_ext.shape[2]
    dh = K_ext.shape[3]
    dcols = hq_per * dh

    Wq_l = lax.dynamic_slice(Wq, (0, my * dcols), (Wq.shape[0], dcols))
    Wo_l = lax.dynamic_slice(Wo, (my * dcols, 0), (dcols, Wo.shape[1]))

    xb = x[0].astype(jnp.bfloat16)
    Q = jnp.dot(xb, Wq_l.astype(jnp.bfloat16), preferred_element_type=jnp.float32)
    q_hm = Q.reshape(sq, hq_per, dh).transpose(1, 0, 2).astype(jnp.bfloat16)
    k_hm = K_ext[0].transpose(1, 0, 2).astype(jnp.bfloat16)
    v_hm = V_ext[0].transpose(1, 0, 2).astype(jnp.bfloat16)

    ctx = _sparse_attn(q_hm, k_hm, v_hm, hq_per)
    ctx = ctx.transpose(1, 0, 2).reshape(sq, dcols)

    partial = jnp.dot(
        ctx, Wo_l.astype(jnp.bfloat16), preferred_element_type=jnp.float32
    )

    out = _ring_allreduce(partial.astype(jnp.bfloat16))
    return out.astype(jnp.float32)[None]
